# Initial kernel scaffold; baseline (speedup 1.0000x reference)
import jax
import jax.numpy as jnp
from jax.experimental import pallas as pl


def kernel(x, edge_index, W_v, W_e, W_upd, b_upd):
    raise NotImplementedError("write your pallas kernel here")


# speedup vs baseline: 4.3303x; 4.3303x over previous
import functools

import jax
import jax.numpy as jnp
from jax import lax
from jax.experimental import pallas as pl
from jax.experimental.pallas import tpu as pltpu
from jax.experimental.pallas import tpu_sc as plsc

NC, NS = 2, 16
NW = NC * NS
CHUNK = 128
D = 128
DA = 144
BLK = 640


def _sc_pass_body(K, stripe, tbl, gidx, sidx, zeros, out, gv, sv, vals, acc, sem):
    cid = lax.axis_index("c")
    sid = lax.axis_index("s")
    wid = cid * NS + sid
    n_pad = tbl.shape[0]
    pltpu.sync_copy(zeros.at[pl.ds(sid * stripe, stripe)],
                    acc.at[pl.ds(sid * stripe, stripe)])
    pltpu.sync_copy(gidx.at[wid], gv)
    pltpu.sync_copy(sidx.at[wid], sv)
    plsc.subcore_barrier()

    def step(j, carry):
        pltpu.async_copy(tbl.at[gv.at[j]], vals, sem).wait()
        pltpu.sync_copy(vals, acc.at[sv.at[j]], add=True)
        return carry

    lax.fori_loop(0, K, step, 0)
    plsc.subcore_barrier()
    pltpu.sync_copy(acc.at[pl.ds(sid * stripe, stripe)],
                    out.at[pl.ds(cid * n_pad + sid * stripe, stripe)])


def _make_sc_pass(n_pad, K):
    stripe = n_pad // NS
    mesh = plsc.VectorSubcoreMesh(core_axis_name="c", subcore_axis_name="s",
                                  num_cores=NC, num_subcores=NS)
    return pl.kernel(
        functools.partial(_sc_pass_body, K, stripe),
        out_type=jax.ShapeDtypeStruct((NC * n_pad, DA), jnp.float32),
        mesh=mesh,
        scratch_types=[
            pltpu.VMEM((K, CHUNK), jnp.int32),
            pltpu.VMEM((K, CHUNK), jnp.int32),
            pltpu.VMEM((CHUNK, DA), jnp.float32),
            pltpu.VMEM_SHARED((n_pad, DA), jnp.float32),
            pltpu.SemaphoreType.DMA,
        ],
        compiler_params=pltpu.CompilerParams(use_tc_tiling_on_sc=False),
    )


def _ones_col(rows):
    return (lax.broadcasted_iota(jnp.int32, (rows, DA - D), 1) == 0).astype(jnp.float32)


def _k1_body(x_ref, w_ref, o_ref):
    m = jnp.dot(x_ref[...], w_ref[...], preferred_element_type=jnp.float32)
    o_ref[...] = jnp.concatenate([m, _ones_col(m.shape[0])], axis=1)


def _k2_body(acc_ref, w_ref, o_ref):
    p = acc_ref[0] + acc_ref[1]
    ef = p[:, :D] / jnp.maximum(p[:, D:D + 1], 1.0)
    ep = jnp.dot(ef, w_ref[...], preferred_element_type=jnp.float32)
    o_ref[...] = jnp.concatenate([ep, _ones_col(ep.shape[0])], axis=1)


def _k3_body(xa_ref, acc_ref, wu_ref, b_ref, o_ref):
    p = acc_ref[0] + acc_ref[1]
    nagg = p[:, :D] / jnp.maximum(p[:, D:D + 1], 1.0)
    h = (jnp.dot(xa_ref[:, :D], wu_ref[:D], preferred_element_type=jnp.float32)
         + jnp.dot(nagg, wu_ref[D:], preferred_element_type=jnp.float32)
         + b_ref[...])
    o_ref[...] = jnp.maximum(h, 0.0)


def kernel(x, edge_index, W_v, W_e, W_upd, b_upd):
    n = x.shape[0]
    e = edge_index.shape[1]
    n_pad = ((n + 1 + BLK - 1) // BLK) * BLK
    K = -(-e // (NW * CHUNK))
    e_pad = NW * K * CHUNK

    row = edge_index[0]
    col = edge_index[1]
    fill = jnp.full((e_pad - e,), n, jnp.int32)
    row_p = jnp.concatenate([row, fill]).reshape(NW, K, CHUNK)
    col_p = jnp.concatenate([col, fill]).reshape(NW, K, CHUNK)

    x_pad = jnp.zeros((n_pad, D), jnp.float32).at[:n].set(x)
    zeros = jnp.zeros((n_pad, DA), jnp.float32)

    grid = n_pad // BLK
    full = lambda shape: pl.BlockSpec(shape, lambda i: (0,) * len(shape))

    x_self_aug = pl.pallas_call(
        _k1_body,
        grid=(grid,),
        in_specs=[pl.BlockSpec((BLK, D), lambda i: (i, 0)), full((D, D))],
        out_specs=pl.BlockSpec((BLK, DA), lambda i: (i, 0)),
        out_shape=jax.ShapeDtypeStruct((n_pad, DA), jnp.float32),
    )(x_pad, W_v)

    sc_pass = _make_sc_pass(n_pad, K)
    acc_a = sc_pass(x_self_aug, row_p, col_p, zeros).reshape(NC, n_pad, DA)

    e_proj_aug = pl.pallas_call(
        _k2_body,
        grid=(grid,),
        in_specs=[pl.BlockSpec((NC, BLK, DA), lambda i: (0, i, 0)), full((D, D))],
        out_specs=pl.BlockSpec((BLK, DA), lambda i: (i, 0)),
        out_shape=jax.ShapeDtypeStruct((n_pad, DA), jnp.float32),
    )(acc_a, W_e)

    acc_b = sc_pass(e_proj_aug, col_p, row_p, zeros).reshape(NC, n_pad, DA)

    out = pl.pallas_call(
        _k3_body,
        grid=(grid,),
        in_specs=[
            pl.BlockSpec((BLK, DA), lambda i: (i, 0)),
            pl.BlockSpec((NC, BLK, DA), lambda i: (0, i, 0)),
            full((2 * D, D)),
            full((1, D)),
        ],
        out_specs=pl.BlockSpec((BLK, D), lambda i: (i, 0)),
        out_shape=jax.ShapeDtypeStruct((n_pad, D), jnp.float32),
    )(x_self_aug, acc_b, W_upd, b_upd.reshape(1, D))

    return out[:n]


# speedup vs baseline: 4.5161x; 1.0429x over previous
import functools

import jax
import jax.numpy as jnp
from jax import lax
from jax.experimental import pallas as pl
from jax.experimental.pallas import tpu as pltpu
from jax.experimental.pallas import tpu_sc as plsc

NC, NS = 2, 16
NW = NC * NS
CHUNK = 64
D = 128
DA = 144
BLK = 640


def _sc_pass_body(K, stripe, tbl, gidx, sidx, zeros, out, gv, sv, vals0, vals1,
                  acc, sem):
    cid = lax.axis_index("c")
    sid = lax.axis_index("s")
    wid = cid * NS + sid
    n_acc = acc.shape[0]
    pltpu.sync_copy(zeros.at[pl.ds(sid * stripe, stripe)],
                    acc.at[pl.ds(sid * stripe, stripe)])
    pltpu.sync_copy(gidx.at[wid], gv)
    pltpu.sync_copy(sidx.at[wid], sv)
    plsc.subcore_barrier()

    pltpu.async_copy(tbl.at[gv.at[0]], vals0, sem)

    def step(i, carry):
        j = 2 * i
        pltpu.make_async_copy(tbl.at[gv.at[j]], vals0, sem).wait()
        pltpu.async_copy(tbl.at[gv.at[j + 1]], vals1, sem)
        pltpu.sync_copy(vals0, acc.at[sv.at[j]], add=True)
        pltpu.make_async_copy(tbl.at[gv.at[j + 1]], vals1, sem).wait()

        @pl.when(j + 2 < K)
        def _():
            pltpu.async_copy(tbl.at[gv.at[j + 2]], vals0, sem)

        pltpu.sync_copy(vals1, acc.at[sv.at[j + 1]], add=True)
        return carry

    lax.fori_loop(0, K // 2, step, 0)
    plsc.subcore_barrier()
    pltpu.sync_copy(acc.at[pl.ds(sid * stripe, stripe)],
                    out.at[pl.ds(cid * n_acc + sid * stripe, stripe)])


def _make_sc_pass(n_acc, K):
    stripe = n_acc // NS
    mesh = plsc.VectorSubcoreMesh(core_axis_name="c", subcore_axis_name="s",
                                  num_cores=NC, num_subcores=NS)
    return pl.kernel(
        functools.partial(_sc_pass_body, K, stripe),
        out_type=jax.ShapeDtypeStruct((NC * n_acc, DA), jnp.float32),
        mesh=mesh,
        scratch_types=[
            pltpu.VMEM((K, CHUNK), jnp.int32),
            pltpu.VMEM((K, CHUNK), jnp.int32),
            pltpu.VMEM((CHUNK, DA), jnp.float32),
            pltpu.VMEM((CHUNK, DA), jnp.float32),
            pltpu.VMEM_SHARED((n_acc, DA), jnp.float32),
            pltpu.SemaphoreType.DMA,
        ],
        compiler_params=pltpu.CompilerParams(use_tc_tiling_on_sc=False),
    )


def _ones_col(rows):
    return (lax.broadcasted_iota(jnp.int32, (rows, DA - D), 1) == 0).astype(jnp.float32)


def _k1_body(x_ref, w_ref, o_ref):
    m = jnp.dot(x_ref[...], w_ref[...], preferred_element_type=jnp.float32)
    o_ref[...] = jnp.concatenate([m, _ones_col(m.shape[0])], axis=1)


def _k2_body(acc_ref, w_ref, o_ref):
    p = acc_ref[0] + acc_ref[1]
    ef = p[:, :D] / jnp.maximum(p[:, D:D + 1], 1.0)
    ep = jnp.dot(ef, w_ref[...], preferred_element_type=jnp.float32)
    o_ref[...] = jnp.concatenate([ep, _ones_col(ep.shape[0])], axis=1)


def _k3_body(xa_ref, acc_ref, wu_ref, b_ref, o_ref):
    p = acc_ref[0] + acc_ref[1]
    nagg = p[:, :D] / jnp.maximum(p[:, D:D + 1], 1.0)
    h = (jnp.dot(xa_ref[:, :D], wu_ref[:D], preferred_element_type=jnp.float32)
         + jnp.dot(nagg, wu_ref[D:], preferred_element_type=jnp.float32)
         + b_ref[...])
    o_ref[...] = jnp.maximum(h, 0.0)


def kernel(x, edge_index, W_v, W_e, W_upd, b_upd):
    n = x.shape[0]
    e = edge_index.shape[1]
    n_pad = ((n + 1 + BLK - 1) // BLK) * BLK
    n_acc = ((n + 1 + NS - 1) // NS) * NS
    K = -(-e // (NW * CHUNK))
    K += K % 2
    e_pad = NW * K * CHUNK

    row = edge_index[0]
    col = edge_index[1]
    fill = jnp.full((e_pad - e,), n, jnp.int32)
    row_p = jnp.concatenate([row, fill]).reshape(NW, K, CHUNK)
    col_p = jnp.concatenate([col, fill]).reshape(NW, K, CHUNK)

    x_pad = jnp.zeros((n_pad, D), jnp.float32).at[:n].set(x)
    zeros = jnp.zeros((n_acc, DA), jnp.float32)

    grid = n_pad // BLK
    full = lambda shape: pl.BlockSpec(shape, lambda i: (0,) * len(shape))

    x_self_aug = pl.pallas_call(
        _k1_body,
        grid=(grid,),
        in_specs=[pl.BlockSpec((BLK, D), lambda i: (i, 0)), full((D, D))],
        out_specs=pl.BlockSpec((BLK, DA), lambda i: (i, 0)),
        out_shape=jax.ShapeDtypeStruct((n_pad, DA), jnp.float32),
    )(x_pad, W_v)

    sc_pass = _make_sc_pass(n_acc, K)
    acc_a = sc_pass(x_self_aug, row_p, col_p, zeros).reshape(NC, n_acc, DA)

    e_proj_aug = pl.pallas_call(
        _k2_body,
        grid=(grid,),
        in_specs=[pl.BlockSpec((NC, BLK, DA), lambda i: (0, i, 0)), full((D, D))],
        out_specs=pl.BlockSpec((BLK, DA), lambda i: (i, 0)),
        out_shape=jax.ShapeDtypeStruct((n_pad, DA), jnp.float32),
    )(acc_a, W_e)

    acc_b = sc_pass(e_proj_aug, col_p, row_p, zeros).reshape(NC, n_acc, DA)

    out = pl.pallas_call(
        _k3_body,
        grid=(grid,),
        in_specs=[
            pl.BlockSpec((BLK, DA), lambda i: (i, 0)),
            pl.BlockSpec((NC, BLK, DA), lambda i: (0, i, 0)),
            full((2 * D, D)),
            full((1, D)),
        ],
        out_specs=pl.BlockSpec((BLK, D), lambda i: (i, 0)),
        out_shape=jax.ShapeDtypeStruct((n_pad, D), jnp.float32),
    )(x_self_aug, acc_b, W_upd, b_upd.reshape(1, D))

    return out[:n]
